# Initial kernel scaffold; baseline (speedup 1.0000x reference)
import functools

import jax
import jax.numpy as jnp
from jax import lax
from jax.experimental import pallas as pl
from jax.experimental.pallas import tpu as pltpu
from jax.experimental.pallas import tpu_sc as plsc

N = 10000
E = 320000
IN_F = 128
H = 4
F = 64
HF = H * F
NEG = 0.2

NC = 2
NS = 16
NW = NC * NS
EPW = E // NW
CH = 80
NCHUNK = EPW // CH
NPAD = 10048
RPT = NPAD // NS


def _proj_body(x_ref, w_ref, a_ref, feat_ref, eler_ref):
    feat = jnp.dot(x_ref[...], w_ref[...], preferred_element_type=jnp.float32)
    feat_ref[...] = feat
    eler_ref[...] = jnp.dot(feat, a_ref[...], preferred_element_type=jnp.float32)


def _project(x, W, A_lr):
    blk = 1000
    return pl.pallas_call(
        _proj_body,
        grid=(N // blk,),
        in_specs=[
            pl.BlockSpec((blk, IN_F), lambda i: (i, 0)),
            pl.BlockSpec((IN_F, HF), lambda i: (0, 0)),
            pl.BlockSpec((HF, 8), lambda i: (0, 0)),
        ],
        out_specs=[
            pl.BlockSpec((blk, HF), lambda i: (i, 0)),
            pl.BlockSpec((blk, 8), lambda i: (i, 0)),
        ],
        out_shape=[
            jax.ShapeDtypeStruct((N, HF), jnp.float32),
            jax.ShapeDtypeStruct((N, 8), jnp.float32),
        ],
    )(x, W, A_lr)


def _edge_logits_body(eler_hbm, src_hbm, dst_hbm, eexp_hbm, dpart_hbm,
                      eler_v, src_v, dst_v, eexp4_v, eexp16_v, zb_v, denom_sh):
    c = lax.axis_index("c")
    s = lax.axis_index("s")
    wid = c * NS + s
    ebase = wid * EPW

    pltpu.sync_copy(eler_hbm, eler_v)

    z16 = jnp.zeros((16,), jnp.float32)

    @pl.loop(0, CH)
    def _(i):
        eexp16_v[i, :] = z16

    @pl.loop(0, RPT)
    def _(i):
        zb_v[i, :] = z16

    pltpu.sync_copy(zb_v, denom_sh.at[pl.ds(s * RPT, RPT)])
    plsc.subcore_barrier()

    @pl.loop(0, NCHUNK)
    def _(k):
        base = ebase + k * CH
        pltpu.sync_copy(src_hbm.at[pl.ds(base, CH)], src_v)
        pltpu.sync_copy(dst_hbm.at[pl.ds(base, CH)], dst_v)

        @pl.loop(0, CH, step=16)
        def _(g):
            rows = lax.iota(jnp.int32, 16) + g
            s16 = src_v[pl.ds(g, 16)]
            d16 = dst_v[pl.ds(g, 16)]
            for h in range(H):
                hv = jnp.full((16,), h, jnp.int32)
                hv4 = jnp.full((16,), h + 4, jnp.int32)
                el = plsc.load_gather(eler_v, [s16, hv])
                er = plsc.load_gather(eler_v, [d16, hv4])
                e = el + er
                e = jnp.maximum(e, NEG * e)
                ex = jnp.exp(e)
                plsc.store_scatter(eexp4_v, [rows, hv], ex)
                plsc.store_scatter(eexp16_v, [rows, hv], ex)

        pltpu.sync_copy(eexp16_v, denom_sh.at[dst_v], add=True)
        pltpu.sync_copy(eexp4_v, eexp_hbm.at[pl.ds(base, CH)])

    plsc.subcore_barrier()
    pltpu.sync_copy(denom_sh.at[pl.ds(s * RPT, RPT)],
                    dpart_hbm.at[c, pl.ds(s * RPT, RPT)])


def _edge_logits(eler, src, dst):
    mesh = plsc.VectorSubcoreMesh(core_axis_name="c", subcore_axis_name="s")
    return pl.kernel(
        _edge_logits_body,
        out_type=[
            jax.ShapeDtypeStruct((E, H), jnp.float32),
            jax.ShapeDtypeStruct((NC, NPAD, 16), jnp.float32),
        ],
        mesh=mesh,
        scratch_types=[
            pltpu.VMEM((N, 8), jnp.float32),
            pltpu.VMEM((CH,), jnp.int32),
            pltpu.VMEM((CH,), jnp.int32),
            pltpu.VMEM((CH, H), jnp.float32),
            pltpu.VMEM((CH, 16), jnp.float32),
            pltpu.VMEM((RPT, 16), jnp.float32),
            pltpu.VMEM_SHARED((NPAD, 16), jnp.float32),
        ],
    )(eler, src, dst)


def _invd_body(d_ref, o_ref):
    d = d_ref[0, :, :4] + d_ref[1, :, :4]
    o_ref[...] = 1.0 / (d + 1e-9)


def _inv_denom(dpart):
    blk = 1000
    return pl.pallas_call(
        _invd_body,
        grid=(N // blk,),
        in_specs=[pl.BlockSpec((NC, blk, 16), lambda i: (0, i, 0))],
        out_specs=pl.BlockSpec((blk, H), lambda i: (i, 0)),
        out_shape=jax.ShapeDtypeStruct((N, H), jnp.float32),
    )(dpart)


def _aggregate_body(feat_hbm, src_hbm, dst_hbm, eexp_hbm, invd_hbm, opart_hbm,
                    invd_v, src_v, dst_v, eexp_v, alpha_v, fbuf_v, msg_v,
                    zb_v, out_sh):
    c = lax.axis_index("c")
    s = lax.axis_index("s")
    wid = c * NS + s
    ebase = wid * EPW

    pltpu.sync_copy(invd_hbm, invd_v)

    z16 = jnp.zeros((16,), jnp.float32)

    @pl.loop(0, RPT)
    def _(i):
        for ci in range(4):
            zb_v[i, pl.ds(ci * 16, 16)] = z16

    pltpu.sync_copy(zb_v, out_sh.at[pl.ds(s * RPT, RPT)])
    plsc.subcore_barrier()

    @pl.loop(0, NCHUNK)
    def _(k):
        base = ebase + k * CH
        pltpu.sync_copy(src_hbm.at[pl.ds(base, CH)], src_v)
        pltpu.sync_copy(dst_hbm.at[pl.ds(base, CH)], dst_v)
        pltpu.sync_copy(eexp_hbm.at[pl.ds(base, CH)], eexp_v)
        pltpu.sync_copy(feat_hbm.at[src_v], fbuf_v)

        @pl.loop(0, CH, step=16)
        def _(g):
            rows = lax.iota(jnp.int32, 16) + g
            d16 = dst_v[pl.ds(g, 16)]
            for h in range(H):
                hv = jnp.full((16,), h, jnp.int32)
                ex = plsc.load_gather(eexp_v, [rows, hv])
                iv = plsc.load_gather(invd_v, [d16, hv])
                plsc.store_scatter(alpha_v, [rows, hv], ex * iv)

        @pl.loop(0, CH)
        def _(e):
            a0 = alpha_v[e, 0]
            a1 = alpha_v[e, 1]
            a2 = alpha_v[e, 2]
            a3 = alpha_v[e, 3]
            for ci in range(4):
                acc = a0 * fbuf_v[e, pl.ds(0 * F + ci * 16, 16)]
                acc = acc + a1 * fbuf_v[e, pl.ds(1 * F + ci * 16, 16)]
                acc = acc + a2 * fbuf_v[e, pl.ds(2 * F + ci * 16, 16)]
                acc = acc + a3 * fbuf_v[e, pl.ds(3 * F + ci * 16, 16)]
                msg_v[e, pl.ds(ci * 16, 16)] = acc

        pltpu.sync_copy(msg_v, out_sh.at[dst_v], add=True)

    plsc.subcore_barrier()
    pltpu.sync_copy(out_sh.at[pl.ds(s * RPT, RPT)],
                    opart_hbm.at[c, pl.ds(s * RPT, RPT)])


def _aggregate(feat, src, dst, eexp, invd):
    mesh = plsc.VectorSubcoreMesh(core_axis_name="c", subcore_axis_name="s")
    return pl.kernel(
        _aggregate_body,
        out_type=jax.ShapeDtypeStruct((NC, NPAD, F), jnp.float32),
        mesh=mesh,
        scratch_types=[
            pltpu.VMEM((N, H), jnp.float32),
            pltpu.VMEM((CH,), jnp.int32),
            pltpu.VMEM((CH,), jnp.int32),
            pltpu.VMEM((CH, H), jnp.float32),
            pltpu.VMEM((CH, H), jnp.float32),
            pltpu.VMEM((CH, HF), jnp.float32),
            pltpu.VMEM((CH, F), jnp.float32),
            pltpu.VMEM((RPT, F), jnp.float32),
            pltpu.VMEM_SHARED((NPAD, F), jnp.float32),
        ],
    )(feat, src, dst, eexp, invd)


def _final_body(p_ref, b_ref, o_ref):
    sm = (p_ref[0] + p_ref[1]) * (1.0 / H)
    b = b_ref[...]
    mb = (b[:, 0:64] + b[:, 64:128] + b[:, 128:192] + b[:, 192:256]) * (1.0 / H)
    o_ref[...] = jnp.maximum(sm + mb, 0.0)


def _finalize(opart, bias2d):
    blk = 1000
    return pl.pallas_call(
        _final_body,
        grid=(N // blk,),
        in_specs=[
            pl.BlockSpec((NC, blk, F), lambda i: (0, i, 0)),
            pl.BlockSpec((1, HF), lambda i: (0, 0)),
        ],
        out_specs=pl.BlockSpec((blk, F), lambda i: (i, 0)),
        out_shape=jax.ShapeDtypeStruct((N, F), jnp.float32),
    )(opart, bias2d)


def kernel(x, edge_index, W, attn_l, attn_r, bias):
    src = edge_index[0].astype(jnp.int32)
    dst = edge_index[1].astype(jnp.int32)

    eye = jnp.eye(H, dtype=jnp.float32)
    Al = jnp.einsum("hf,hk->hfk", attn_l, eye).reshape(HF, H)
    Ar = jnp.einsum("hf,hk->hfk", attn_r, eye).reshape(HF, H)
    A_lr = jnp.concatenate([Al, Ar], axis=1)

    feat, eler = _project(x, W, A_lr)
    eexp, dpart = _edge_logits(eler, src, dst)
    invd = _inv_denom(dpart[:, :N, :])
    opart = _aggregate(feat, src, dst, eexp, invd)
    return _finalize(opart[:, :N, :], bias.reshape(1, HF))


# speedup vs baseline: 21.1337x; 21.1337x over previous
import dataclasses
import functools

import jax
import jax.numpy as jnp
from jax import lax
from jax.experimental import pallas as pl
from jax.experimental.pallas import tpu as pltpu
from jax.experimental.pallas import tpu_sc as plsc

N = 10000
E = 320000
IN_F = 128
H = 4
F = 64
HF = H * F
NEG = 0.2

NC = 2
NS = 16
NW = NC * NS
EPW = E // NW
CH = 80
NCHUNK = EPW // CH
NPAD = 10240
RPT = NPAD // NS


def _sc_compiler_params():
    return pltpu.CompilerParams(
        needs_layout_passes=False, use_tc_tiling_on_sc=False
    )


def _proj_body(x_ref, w_ref, a_ref, feat_ref, eler_ref):
    feat = jnp.dot(x_ref[...], w_ref[...], preferred_element_type=jnp.float32)
    feat_ref[...] = feat
    eler_ref[...] = jnp.dot(feat, a_ref[...], preferred_element_type=jnp.float32)


def _project(x, W, A_lr):
    blk = 1000
    return pl.pallas_call(
        _proj_body,
        grid=(N // blk,),
        in_specs=[
            pl.BlockSpec((blk, IN_F), lambda i: (i, 0)),
            pl.BlockSpec((IN_F, HF), lambda i: (0, 0)),
            pl.BlockSpec((HF, 16), lambda i: (0, 0)),
        ],
        out_specs=[
            pl.BlockSpec((blk, HF), lambda i: (i, 0)),
            pl.BlockSpec((blk, 16), lambda i: (i, 0)),
        ],
        out_shape=[
            jax.ShapeDtypeStruct((N, HF), jnp.float32),
            jax.ShapeDtypeStruct((N, 16), jnp.float32),
        ],
    )(x, W, A_lr)


def _edge_logits_body(eler_hbm, src_hbm, dst_hbm, eexp_hbm, dpart_hbm,
                      els_v, erd_v, src_v, dst_v, eexp4_v, eexp16_v, denom_sh):
    c = lax.axis_index("c")
    s = lax.axis_index("s")
    wid = c * NS + s
    ebase = wid * EPW

    z16 = jnp.zeros((16,), jnp.float32)

    @pl.loop(0, CH)
    def _(i):
        eexp16_v[i, :] = z16

    @pl.loop(0, RPT // CH)
    def _(j):
        pltpu.sync_copy(eexp16_v, denom_sh.at[pl.ds(s * RPT + j * CH, CH)])

    plsc.subcore_barrier()

    @pl.loop(0, NCHUNK)
    def _(k):
        base = ebase + k * CH
        pltpu.sync_copy(src_hbm.at[pl.ds(base, CH)], src_v)
        pltpu.sync_copy(dst_hbm.at[pl.ds(base, CH)], dst_v)
        pltpu.sync_copy(eler_hbm.at[src_v], els_v)
        pltpu.sync_copy(eler_hbm.at[dst_v], erd_v)

        @pl.loop(0, CH, step=16)
        def _(g):
            rows = lax.iota(jnp.int32, 16) + g
            for h in range(H):
                hv = jnp.full((16,), h, jnp.int32)
                hv4 = jnp.full((16,), h + 4, jnp.int32)
                el = plsc.load_gather(els_v, [rows, hv])
                er = plsc.load_gather(erd_v, [rows, hv4])
                e = el + er
                e = jnp.maximum(e, NEG * e)
                ex = jnp.exp(e)
                plsc.store_scatter(eexp4_v, [rows, hv], ex)
                plsc.store_scatter(eexp16_v, [rows, hv], ex)

        pltpu.sync_copy(eexp16_v, denom_sh.at[dst_v], add=True)
        pltpu.sync_copy(eexp4_v, eexp_hbm.at[pl.ds(base, CH)])

    plsc.subcore_barrier()
    pltpu.sync_copy(denom_sh.at[pl.ds(s * RPT, RPT)],
                    dpart_hbm.at[c, pl.ds(s * RPT, RPT)])


def _edge_logits(eler, src, dst):
    mesh = plsc.VectorSubcoreMesh(core_axis_name="c", subcore_axis_name="s")
    return pl.kernel(
        _edge_logits_body,
        out_type=[
            jax.ShapeDtypeStruct((E, H), jnp.float32),
            jax.ShapeDtypeStruct((NC, NPAD, 16), jnp.float32),
        ],
        mesh=mesh,
        scratch_types=[
            pltpu.VMEM((CH, 16), jnp.float32),
            pltpu.VMEM((CH, 16), jnp.float32),
            pltpu.VMEM((CH,), jnp.int32),
            pltpu.VMEM((CH,), jnp.int32),
            pltpu.VMEM((CH, H), jnp.float32),
            pltpu.VMEM((CH, 16), jnp.float32),
            pltpu.VMEM_SHARED((NPAD, 16), jnp.float32),
        ],
        compiler_params=_sc_compiler_params(),
    )(eler, src, dst)


def _invd_body(d_ref, o_ref):
    o_ref[...] = 1.0 / (d_ref[0] + d_ref[1] + 1e-9)


def _inv_denom(dpart):
    blk = 1000
    return pl.pallas_call(
        _invd_body,
        grid=(N // blk,),
        in_specs=[pl.BlockSpec((NC, blk, 16), lambda i: (0, i, 0))],
        out_specs=pl.BlockSpec((blk, 16), lambda i: (i, 0)),
        out_shape=jax.ShapeDtypeStruct((N, 16), jnp.float32),
    )(dpart)


def _aggregate_body(feat_hbm, src_hbm, dst_hbm, eexp_hbm, invd_hbm, opart_hbm,
                    ivbuf_v, src_v, dst_v, eexp_v, alpha_v, fbuf_v, msg_v,
                    out_sh):
    c = lax.axis_index("c")
    s = lax.axis_index("s")
    wid = c * NS + s
    ebase = wid * EPW

    z16 = jnp.zeros((16,), jnp.float32)

    @pl.loop(0, CH)
    def _(i):
        for ci in range(4):
            msg_v[i, pl.ds(ci * 16, 16)] = z16

    @pl.loop(0, RPT // CH)
    def _(j):
        pltpu.sync_copy(msg_v, out_sh.at[pl.ds(s * RPT + j * CH, CH)])

    plsc.subcore_barrier()

    @pl.loop(0, NCHUNK)
    def _(k):
        base = ebase + k * CH
        pltpu.sync_copy(src_hbm.at[pl.ds(base, CH)], src_v)
        pltpu.sync_copy(dst_hbm.at[pl.ds(base, CH)], dst_v)
        pltpu.sync_copy(eexp_hbm.at[pl.ds(base, CH)], eexp_v)
        pltpu.sync_copy(invd_hbm.at[dst_v], ivbuf_v)
        pltpu.sync_copy(feat_hbm.at[src_v], fbuf_v)

        @pl.loop(0, CH, step=16)
        def _(g):
            rows = lax.iota(jnp.int32, 16) + g
            for h in range(H):
                hv = jnp.full((16,), h, jnp.int32)
                ex = plsc.load_gather(eexp_v, [rows, hv])
                iv = plsc.load_gather(ivbuf_v, [rows, hv])
                plsc.store_scatter(alpha_v, [rows, hv], ex * iv)

        @pl.loop(0, CH)
        def _(e):
            arow = alpha_v[e, :]
            a0 = arow[0]
            a1 = arow[1]
            a2 = arow[2]
            a3 = arow[3]
            for ci in range(4):
                acc = a0 * fbuf_v[e, pl.ds(0 * F + ci * 16, 16)]
                acc = acc + a1 * fbuf_v[e, pl.ds(1 * F + ci * 16, 16)]
                acc = acc + a2 * fbuf_v[e, pl.ds(2 * F + ci * 16, 16)]
                acc = acc + a3 * fbuf_v[e, pl.ds(3 * F + ci * 16, 16)]
                msg_v[e, pl.ds(ci * 16, 16)] = acc

        pltpu.sync_copy(msg_v, out_sh.at[dst_v], add=True)

    plsc.subcore_barrier()
    pltpu.sync_copy(out_sh.at[pl.ds(s * RPT, RPT)],
                    opart_hbm.at[c, pl.ds(s * RPT, RPT)])


def _aggregate(feat, src, dst, eexp, invd):
    mesh = plsc.VectorSubcoreMesh(core_axis_name="c", subcore_axis_name="s")
    return pl.kernel(
        _aggregate_body,
        out_type=jax.ShapeDtypeStruct((NC, NPAD, F), jnp.float32),
        mesh=mesh,
        scratch_types=[
            pltpu.VMEM((CH, 16), jnp.float32),
            pltpu.VMEM((CH,), jnp.int32),
            pltpu.VMEM((CH,), jnp.int32),
            pltpu.VMEM((CH, H), jnp.float32),
            pltpu.VMEM((CH, 16), jnp.float32),
            pltpu.VMEM((CH, HF), jnp.float32),
            pltpu.VMEM((CH, F), jnp.float32),
            pltpu.VMEM_SHARED((NPAD, F), jnp.float32),
        ],
        compiler_params=_sc_compiler_params(),
    )(feat, src, dst, eexp, invd)


def _final_body(p_ref, b_ref, o_ref):
    sm = (p_ref[0] + p_ref[1]) * (1.0 / H)
    b = b_ref[...]
    mb = (b[:, 0:64] + b[:, 64:128] + b[:, 128:192] + b[:, 192:256]) * (1.0 / H)
    o_ref[...] = jnp.maximum(sm + mb, 0.0)


def _finalize(opart, bias2d):
    blk = 1000
    return pl.pallas_call(
        _final_body,
        grid=(N // blk,),
        in_specs=[
            pl.BlockSpec((NC, blk, F), lambda i: (0, i, 0)),
            pl.BlockSpec((1, HF), lambda i: (0, 0)),
        ],
        out_specs=pl.BlockSpec((blk, F), lambda i: (i, 0)),
        out_shape=jax.ShapeDtypeStruct((N, F), jnp.float32),
    )(opart, bias2d)


def kernel(x, edge_index, W, attn_l, attn_r, bias):
    src = edge_index[0].astype(jnp.int32)
    dst = edge_index[1].astype(jnp.int32)

    eye = jnp.eye(H, dtype=jnp.float32)
    Al = jnp.einsum("hf,hk->hfk", attn_l, eye).reshape(HF, H)
    Ar = jnp.einsum("hf,hk->hfk", attn_r, eye).reshape(HF, H)
    A_lr = jnp.concatenate(
        [Al, Ar, jnp.zeros((HF, 8), jnp.float32)], axis=1)

    feat, eler = _project(x, W, A_lr)
    eexp, dpart = _edge_logits(eler, src, dst)
    invd = _inv_denom(dpart[:, :N, :])
    opart = _aggregate(feat, src, dst, eexp, invd)
    return _finalize(opart[:, :N, :], bias.reshape(1, HF))


# speedup vs baseline: 29.6913x; 1.4049x over previous
import dataclasses
import functools

import jax
import jax.numpy as jnp
from jax import lax
from jax.experimental import pallas as pl
from jax.experimental.pallas import tpu as pltpu
from jax.experimental.pallas import tpu_sc as plsc

N = 10000
E = 320000
IN_F = 128
H = 4
F = 64
HF = H * F
NEG = 0.2

NC = 2
NS = 16
NW = NC * NS
EPW = E // NW
CH = 80
NCHUNK = EPW // CH
NPAD = 10240
RPT = NPAD // NS


def _sc_compiler_params():
    return pltpu.CompilerParams(
        needs_layout_passes=False, use_tc_tiling_on_sc=False
    )


def _proj_body(x_ref, w_ref, a_ref, feat_ref, eler_ref):
    feat = jnp.dot(x_ref[...], w_ref[...], preferred_element_type=jnp.float32)
    feat_ref[...] = feat
    eler_ref[...] = jnp.dot(feat, a_ref[...], preferred_element_type=jnp.float32)


def _project(x, W, A_lr):
    blk = 1000
    return pl.pallas_call(
        _proj_body,
        grid=(N // blk,),
        in_specs=[
            pl.BlockSpec((blk, IN_F), lambda i: (i, 0)),
            pl.BlockSpec((IN_F, HF), lambda i: (0, 0)),
            pl.BlockSpec((HF, 16), lambda i: (0, 0)),
        ],
        out_specs=[
            pl.BlockSpec((blk, HF), lambda i: (i, 0)),
            pl.BlockSpec((blk, 16), lambda i: (i, 0)),
        ],
        out_shape=[
            jax.ShapeDtypeStruct((N, HF), jnp.float32),
            jax.ShapeDtypeStruct((N, 16), jnp.float32),
        ],
    )(x, W, A_lr)


def _edge_logits_body(eler_hbm, src_hbm, dst_hbm, eexp_hbm, dpart_hbm,
                      els_v, erd_v, src_v, dst_v, eexp4_v, eexp16_v, denom_sh):
    c = lax.axis_index("c")
    s = lax.axis_index("s")
    wid = c * NS + s
    ebase = wid * EPW

    z16 = jnp.zeros((16,), jnp.float32)

    @pl.loop(0, CH)
    def _(i):
        eexp16_v[i, :] = z16

    @pl.loop(0, RPT // CH)
    def _(j):
        pltpu.sync_copy(eexp16_v, denom_sh.at[pl.ds(s * RPT + j * CH, CH)])

    plsc.subcore_barrier()

    @pl.loop(0, NCHUNK)
    def _(k):
        base = ebase + k * CH
        pltpu.sync_copy(src_hbm.at[pl.ds(base, CH)], src_v)
        pltpu.sync_copy(dst_hbm.at[pl.ds(base, CH)], dst_v)
        pltpu.sync_copy(eler_hbm.at[src_v], els_v)
        pltpu.sync_copy(eler_hbm.at[dst_v], erd_v)

        @pl.loop(0, CH, step=16)
        def _(g):
            rows = lax.iota(jnp.int32, 16) + g
            for h in range(H):
                hv = jnp.full((16,), h, jnp.int32)
                hv4 = jnp.full((16,), h + 4, jnp.int32)
                el = plsc.load_gather(els_v, [rows, hv])
                er = plsc.load_gather(erd_v, [rows, hv4])
                e = el + er
                e = jnp.maximum(e, NEG * e)
                ex = jnp.exp(e)
                plsc.store_scatter(eexp4_v, [rows, hv], ex)
                plsc.store_scatter(eexp16_v, [rows, hv], ex)

        pltpu.sync_copy(eexp16_v, denom_sh.at[dst_v], add=True)
        pltpu.sync_copy(eexp4_v, eexp_hbm.at[pl.ds(base, CH)])

    plsc.subcore_barrier()
    pltpu.sync_copy(denom_sh.at[pl.ds(s * RPT, RPT)],
                    dpart_hbm.at[c, pl.ds(s * RPT, RPT)])


def _edge_logits(eler, src, dst):
    mesh = plsc.VectorSubcoreMesh(core_axis_name="c", subcore_axis_name="s")
    return pl.kernel(
        _edge_logits_body,
        out_type=[
            jax.ShapeDtypeStruct((E, H), jnp.float32),
            jax.ShapeDtypeStruct((NC, NPAD, 16), jnp.float32),
        ],
        mesh=mesh,
        scratch_types=[
            pltpu.VMEM((CH, 16), jnp.float32),
            pltpu.VMEM((CH, 16), jnp.float32),
            pltpu.VMEM((CH,), jnp.int32),
            pltpu.VMEM((CH,), jnp.int32),
            pltpu.VMEM((CH, H), jnp.float32),
            pltpu.VMEM((CH, 16), jnp.float32),
            pltpu.VMEM_SHARED((NPAD, 16), jnp.float32),
        ],
        compiler_params=_sc_compiler_params(),
    )(eler, src, dst)


def _invd_body(d_ref, o_ref):
    o_ref[...] = 1.0 / (d_ref[0] + d_ref[1] + 1e-9)


def _inv_denom(dpart):
    blk = 1000
    return pl.pallas_call(
        _invd_body,
        grid=(N // blk,),
        in_specs=[pl.BlockSpec((NC, blk, 16), lambda i: (0, i, 0))],
        out_specs=pl.BlockSpec((blk, 16), lambda i: (i, 0)),
        out_shape=jax.ShapeDtypeStruct((N, 16), jnp.float32),
    )(dpart)


def _aggregate_body(feat_hbm, src_hbm, dst_hbm, eexp_hbm, invd_hbm, opart_hbm,
                    iv0_v, iv1_v, src0_v, src1_v, dst0_v, dst1_v,
                    eexp0_v, eexp1_v, alpha_v, fbuf0_v, fbuf1_v, msg_v,
                    out_sh, semi0, semi1, semg0, semg1):
    c = lax.axis_index("c")
    s = lax.axis_index("s")
    wid = c * NS + s
    ebase = wid * EPW

    srcv = (src0_v, src1_v)
    dstv = (dst0_v, dst1_v)
    eexpv = (eexp0_v, eexp1_v)
    fbufv = (fbuf0_v, fbuf1_v)
    ivv = (iv0_v, iv1_v)
    semi = (semi0, semi1)
    semg = (semg0, semg1)

    def idx_copies(k, b):
        base = ebase + k * CH
        return (
            pltpu.make_async_copy(src_hbm.at[pl.ds(base, CH)], srcv[b], semi[b]),
            pltpu.make_async_copy(dst_hbm.at[pl.ds(base, CH)], dstv[b], semi[b]),
            pltpu.make_async_copy(eexp_hbm.at[pl.ds(base, CH)], eexpv[b], semi[b]),
        )

    def gather_copies(b):
        return (
            pltpu.make_async_copy(feat_hbm.at[srcv[b]], fbufv[b], semg[b]),
            pltpu.make_async_copy(invd_hbm.at[dstv[b]], ivv[b], semg[b]),
        )

    def issue(copies):
        for cp in copies:
            cp.start()

    def wait(copies):
        for cp in copies:
            cp.wait()

    def compute_and_scatter(b):
        @pl.loop(0, CH, step=16)
        def _(g):
            rows = lax.iota(jnp.int32, 16) + g
            for h in range(H):
                hv = jnp.full((16,), h, jnp.int32)
                ex = plsc.load_gather(eexpv[b], [rows, hv])
                iv = plsc.load_gather(ivv[b], [rows, hv])
                plsc.store_scatter(alpha_v, [rows, hv], ex * iv)

        @pl.loop(0, CH)
        def _(e):
            arow = alpha_v[e, :]
            a0 = arow[0]
            a1 = arow[1]
            a2 = arow[2]
            a3 = arow[3]
            for ci in range(4):
                acc = a0 * fbufv[b][e, pl.ds(0 * F + ci * 16, 16)]
                acc = acc + a1 * fbufv[b][e, pl.ds(1 * F + ci * 16, 16)]
                acc = acc + a2 * fbufv[b][e, pl.ds(2 * F + ci * 16, 16)]
                acc = acc + a3 * fbufv[b][e, pl.ds(3 * F + ci * 16, 16)]
                msg_v[e, pl.ds(ci * 16, 16)] = acc

        pltpu.sync_copy(msg_v, out_sh.at[dstv[b]], add=True)

    z16 = jnp.zeros((16,), jnp.float32)

    @pl.loop(0, CH)
    def _(i):
        for ci in range(4):
            msg_v[i, pl.ds(ci * 16, 16)] = z16

    @pl.loop(0, RPT // CH)
    def _(j):
        pltpu.sync_copy(msg_v, out_sh.at[pl.ds(s * RPT + j * CH, CH)])

    plsc.subcore_barrier()

    issue(idx_copies(0, 0))
    wait(idx_copies(0, 0))
    issue(gather_copies(0))
    issue(idx_copies(1, 1))

    @pl.loop(0, NCHUNK - 1, step=2)
    def _(k):
        for b in (0, 1):
            kk = k + b
            wait(gather_copies(b))
            wait(idx_copies(kk + 1, 1 - b))
            issue(gather_copies(1 - b))
            compute_and_scatter(b)

            @pl.when(kk + 2 < NCHUNK)
            def _():
                issue(idx_copies(kk + 2, b))

    wait(gather_copies(0))
    compute_and_scatter(0)

    plsc.subcore_barrier()
    pltpu.sync_copy(out_sh.at[pl.ds(s * RPT, RPT)],
                    opart_hbm.at[c, pl.ds(s * RPT, RPT)])


def _aggregate(feat, src, dst, eexp, invd):
    mesh = plsc.VectorSubcoreMesh(core_axis_name="c", subcore_axis_name="s")
    return pl.kernel(
        _aggregate_body,
        out_type=jax.ShapeDtypeStruct((NC, NPAD, F), jnp.float32),
        mesh=mesh,
        scratch_types=[
            pltpu.VMEM((CH, 16), jnp.float32),
            pltpu.VMEM((CH, 16), jnp.float32),
            pltpu.VMEM((CH,), jnp.int32),
            pltpu.VMEM((CH,), jnp.int32),
            pltpu.VMEM((CH,), jnp.int32),
            pltpu.VMEM((CH,), jnp.int32),
            pltpu.VMEM((CH, H), jnp.float32),
            pltpu.VMEM((CH, H), jnp.float32),
            pltpu.VMEM((CH, 16), jnp.float32),
            pltpu.VMEM((CH, HF), jnp.float32),
            pltpu.VMEM((CH, HF), jnp.float32),
            pltpu.VMEM((CH, F), jnp.float32),
            pltpu.VMEM_SHARED((NPAD, F), jnp.float32),
            pltpu.SemaphoreType.DMA,
            pltpu.SemaphoreType.DMA,
            pltpu.SemaphoreType.DMA,
            pltpu.SemaphoreType.DMA,
        ],
        compiler_params=_sc_compiler_params(),
    )(feat, src, dst, eexp, invd)


def _final_body(p_ref, b_ref, o_ref):
    sm = (p_ref[0] + p_ref[1]) * (1.0 / H)
    b = b_ref[...]
    mb = (b[:, 0:64] + b[:, 64:128] + b[:, 128:192] + b[:, 192:256]) * (1.0 / H)
    o_ref[...] = jnp.maximum(sm + mb, 0.0)


def _finalize(opart, bias2d):
    blk = 1000
    return pl.pallas_call(
        _final_body,
        grid=(N // blk,),
        in_specs=[
            pl.BlockSpec((NC, blk, F), lambda i: (0, i, 0)),
            pl.BlockSpec((1, HF), lambda i: (0, 0)),
        ],
        out_specs=pl.BlockSpec((blk, F), lambda i: (i, 0)),
        out_shape=jax.ShapeDtypeStruct((N, F), jnp.float32),
    )(opart, bias2d)


def kernel(x, edge_index, W, attn_l, attn_r, bias):
    src = edge_index[0].astype(jnp.int32)
    dst = edge_index[1].astype(jnp.int32)

    eye = jnp.eye(H, dtype=jnp.float32)
    Al = jnp.einsum("hf,hk->hfk", attn_l, eye).reshape(HF, H)
    Ar = jnp.einsum("hf,hk->hfk", attn_r, eye).reshape(HF, H)
    A_lr = jnp.concatenate(
        [Al, Ar, jnp.zeros((HF, 8), jnp.float32)], axis=1)

    feat, eler = _project(x, W, A_lr)
    eexp, dpart = _edge_logits(eler, src, dst)
    invd = _inv_denom(dpart[:, :N, :])
    opart = _aggregate(feat, src, dst, eexp, invd)
    return _finalize(opart[:, :N, :], bias.reshape(1, HF))


# speedup vs baseline: 38.9538x; 1.3120x over previous
import jax
import jax.numpy as jnp
from jax import lax
from jax.experimental import pallas as pl
from jax.experimental.pallas import tpu as pltpu
from jax.experimental.pallas import tpu_sc as plsc

N = 10000
E = 320000
IN_F = 128
H = 4
F = 64
HF = H * F
NEG = 0.2

NC = 2
NS = 16
NW = NC * NS
EPW = E // NW
CH = 80
NCHUNK = EPW // CH
NPAD = 10240
RPT = NPAD // NS


def _sc_compiler_params():
    return pltpu.CompilerParams(
        needs_layout_passes=False, use_tc_tiling_on_sc=False
    )


def _proj_body(x_ref, w_ref, a_ref, feat_ref, eler_ref):
    feat = jnp.dot(x_ref[...], w_ref[...], preferred_element_type=jnp.float32)
    feat_ref[...] = feat
    eler_ref[...] = jnp.dot(feat, a_ref[...], preferred_element_type=jnp.float32)


def _project(x, W, A_lr):
    blk = 1000
    return pl.pallas_call(
        _proj_body,
        grid=(N // blk,),
        in_specs=[
            pl.BlockSpec((blk, IN_F), lambda i: (i, 0)),
            pl.BlockSpec((IN_F, HF), lambda i: (0, 0)),
            pl.BlockSpec((HF, 16), lambda i: (0, 0)),
        ],
        out_specs=[
            pl.BlockSpec((blk, HF), lambda i: (i, 0)),
            pl.BlockSpec((blk, 16), lambda i: (i, 0)),
        ],
        out_shape=[
            jax.ShapeDtypeStruct((N, HF), jnp.float32),
            jax.ShapeDtypeStruct((N, 16), jnp.float32),
        ],
    )(x, W, A_lr)


def _edge_logits_body(eler_hbm, src_hbm, dst_hbm, eexp_hbm, dpart_hbm,
                      els0_v, els1_v, erd0_v, erd1_v, src0_v, src1_v,
                      dst0_v, dst1_v, eexp4_v, eexp16_v, denom_sh,
                      semi0, semi1, semg0, semg1):
    c = lax.axis_index("c")
    s = lax.axis_index("s")
    wid = c * NS + s
    ebase = wid * EPW

    srcv = (src0_v, src1_v)
    dstv = (dst0_v, dst1_v)
    elsv = (els0_v, els1_v)
    erdv = (erd0_v, erd1_v)
    semi = (semi0, semi1)
    semg = (semg0, semg1)

    def idx_copies(k, b):
        base = ebase + k * CH
        return (
            pltpu.make_async_copy(src_hbm.at[pl.ds(base, CH)], srcv[b], semi[b]),
            pltpu.make_async_copy(dst_hbm.at[pl.ds(base, CH)], dstv[b], semi[b]),
        )

    def gather_copies(b):
        return (
            pltpu.make_async_copy(eler_hbm.at[srcv[b]], elsv[b], semg[b]),
            pltpu.make_async_copy(eler_hbm.at[dstv[b]], erdv[b], semg[b]),
        )

    def issue(copies):
        for cp in copies:
            cp.start()

    def wait(copies):
        for cp in copies:
            cp.wait()

    def compute_and_store(k, b):
        base = ebase + k * CH

        @pl.loop(0, CH, step=16)
        def _(g):
            rows = lax.iota(jnp.int32, 16) + g
            for h in range(H):
                hv = jnp.full((16,), h, jnp.int32)
                hv4 = jnp.full((16,), h + 4, jnp.int32)
                el = plsc.load_gather(elsv[b], [rows, hv])
                er = plsc.load_gather(erdv[b], [rows, hv4])
                e = el + er
                e = jnp.maximum(e, NEG * e)
                ex = jnp.exp(e)
                plsc.store_scatter(eexp4_v, [rows, hv], ex)
                plsc.store_scatter(eexp16_v, [rows, hv], ex)

        pltpu.sync_copy(eexp16_v, denom_sh.at[dstv[b]], add=True)
        pltpu.sync_copy(eexp4_v, eexp_hbm.at[pl.ds(base, CH)])

    z16 = jnp.zeros((16,), jnp.float32)

    @pl.loop(0, CH)
    def _(i):
        eexp16_v[i, :] = z16

    @pl.loop(0, RPT // CH)
    def _(j):
        pltpu.sync_copy(eexp16_v, denom_sh.at[pl.ds(s * RPT + j * CH, CH)])

    plsc.subcore_barrier()

    issue(idx_copies(0, 0))
    wait(idx_copies(0, 0))
    issue(gather_copies(0))
    issue(idx_copies(1, 1))

    @pl.loop(0, NCHUNK - 1, step=2)
    def _(k):
        for b in (0, 1):
            kk = k + b
            wait(gather_copies(b))
            wait(idx_copies(kk + 1, 1 - b))
            issue(gather_copies(1 - b))
            compute_and_store(kk, b)

            @pl.when(kk + 2 < NCHUNK)
            def _():
                issue(idx_copies(kk + 2, b))

    wait(gather_copies(0))
    compute_and_store(NCHUNK - 1, 0)

    plsc.subcore_barrier()
    pltpu.sync_copy(denom_sh.at[pl.ds(s * RPT, RPT)],
                    dpart_hbm.at[c, pl.ds(s * RPT, RPT)])


def _edge_logits(eler, src, dst):
    mesh = plsc.VectorSubcoreMesh(core_axis_name="c", subcore_axis_name="s")
    return pl.kernel(
        _edge_logits_body,
        out_type=[
            jax.ShapeDtypeStruct((E, H), jnp.float32),
            jax.ShapeDtypeStruct((NC, NPAD, 16), jnp.float32),
        ],
        mesh=mesh,
        scratch_types=[
            pltpu.VMEM((CH, 16), jnp.float32),
            pltpu.VMEM((CH, 16), jnp.float32),
            pltpu.VMEM((CH, 16), jnp.float32),
            pltpu.VMEM((CH, 16), jnp.float32),
            pltpu.VMEM((CH,), jnp.int32),
            pltpu.VMEM((CH,), jnp.int32),
            pltpu.VMEM((CH,), jnp.int32),
            pltpu.VMEM((CH,), jnp.int32),
            pltpu.VMEM((CH, H), jnp.float32),
            pltpu.VMEM((CH, 16), jnp.float32),
            pltpu.VMEM_SHARED((NPAD, 16), jnp.float32),
            pltpu.SemaphoreType.DMA,
            pltpu.SemaphoreType.DMA,
            pltpu.SemaphoreType.DMA,
            pltpu.SemaphoreType.DMA,
        ],
        compiler_params=_sc_compiler_params(),
    )(eler, src, dst)


def _invd_body(d_ref, o_ref):
    o_ref[...] = 1.0 / (d_ref[0] + d_ref[1] + 1e-9)


def _inv_denom(dpart):
    blk = 1000
    return pl.pallas_call(
        _invd_body,
        grid=(N // blk,),
        in_specs=[pl.BlockSpec((NC, blk, 16), lambda i: (0, i, 0))],
        out_specs=pl.BlockSpec((blk, 16), lambda i: (i, 0)),
        out_shape=jax.ShapeDtypeStruct((N, 16), jnp.float32),
    )(dpart)


def _aggregate_body(feat_hbm, src_hbm, dst_hbm, eexp_hbm, invd_hbm, opart_hbm,
                    iv0_v, iv1_v, src0_v, src1_v, dst0_v, dst1_v,
                    eexp0_v, eexp1_v, alpha_v, fbuf0_v, fbuf1_v, msg_v,
                    out_sh, semi0, semi1, semg0, semg1):
    c = lax.axis_index("c")
    s = lax.axis_index("s")
    wid = c * NS + s
    ebase = wid * EPW

    srcv = (src0_v, src1_v)
    dstv = (dst0_v, dst1_v)
    eexpv = (eexp0_v, eexp1_v)
    fbufv = (fbuf0_v, fbuf1_v)
    ivv = (iv0_v, iv1_v)
    semi = (semi0, semi1)
    semg = (semg0, semg1)

    def idx_copies(k, b):
        base = ebase + k * CH
        return (
            pltpu.make_async_copy(src_hbm.at[pl.ds(base, CH)], srcv[b], semi[b]),
            pltpu.make_async_copy(dst_hbm.at[pl.ds(base, CH)], dstv[b], semi[b]),
            pltpu.make_async_copy(eexp_hbm.at[pl.ds(base, CH)], eexpv[b], semi[b]),
        )

    def gather_copies(b):
        return (
            pltpu.make_async_copy(feat_hbm.at[srcv[b]], fbufv[b], semg[b]),
            pltpu.make_async_copy(invd_hbm.at[dstv[b]], ivv[b], semg[b]),
        )

    def issue(copies):
        for cp in copies:
            cp.start()

    def wait(copies):
        for cp in copies:
            cp.wait()

    def compute_and_scatter(b):
        @pl.loop(0, CH, step=16)
        def _(g):
            rows = lax.iota(jnp.int32, 16) + g
            for h in range(H):
                hv = jnp.full((16,), h, jnp.int32)
                ex = plsc.load_gather(eexpv[b], [rows, hv])
                iv = plsc.load_gather(ivv[b], [rows, hv])
                plsc.store_scatter(alpha_v, [rows, hv], ex * iv)

        @pl.loop(0, CH)
        def _(e):
            arow = alpha_v[e, :]
            a0 = arow[0]
            a1 = arow[1]
            a2 = arow[2]
            a3 = arow[3]
            for ci in range(4):
                acc = a0 * fbufv[b][e, pl.ds(0 * F + ci * 16, 16)]
                acc = acc + a1 * fbufv[b][e, pl.ds(1 * F + ci * 16, 16)]
                acc = acc + a2 * fbufv[b][e, pl.ds(2 * F + ci * 16, 16)]
                acc = acc + a3 * fbufv[b][e, pl.ds(3 * F + ci * 16, 16)]
                msg_v[e, pl.ds(ci * 16, 16)] = acc

        pltpu.sync_copy(msg_v, out_sh.at[dstv[b]], add=True)

    z16 = jnp.zeros((16,), jnp.float32)

    @pl.loop(0, CH)
    def _(i):
        for ci in range(4):
            msg_v[i, pl.ds(ci * 16, 16)] = z16

    @pl.loop(0, RPT // CH)
    def _(j):
        pltpu.sync_copy(msg_v, out_sh.at[pl.ds(s * RPT + j * CH, CH)])

    plsc.subcore_barrier()

    issue(idx_copies(0, 0))
    wait(idx_copies(0, 0))
    issue(gather_copies(0))
    issue(idx_copies(1, 1))

    @pl.loop(0, NCHUNK - 1, step=2)
    def _(k):
        for b in (0, 1):
            kk = k + b
            wait(gather_copies(b))
            wait(idx_copies(kk + 1, 1 - b))
            issue(gather_copies(1 - b))
            compute_and_scatter(b)

            @pl.when(kk + 2 < NCHUNK)
            def _():
                issue(idx_copies(kk + 2, b))

    wait(gather_copies(0))
    compute_and_scatter(0)

    plsc.subcore_barrier()
    pltpu.sync_copy(out_sh.at[pl.ds(s * RPT, RPT)],
                    opart_hbm.at[c, pl.ds(s * RPT, RPT)])


def _aggregate(feat, src, dst, eexp, invd):
    mesh = plsc.VectorSubcoreMesh(core_axis_name="c", subcore_axis_name="s")
    return pl.kernel(
        _aggregate_body,
        out_type=jax.ShapeDtypeStruct((NC, NPAD, F), jnp.float32),
        mesh=mesh,
        scratch_types=[
            pltpu.VMEM((CH, 16), jnp.float32),
            pltpu.VMEM((CH, 16), jnp.float32),
            pltpu.VMEM((CH,), jnp.int32),
            pltpu.VMEM((CH,), jnp.int32),
            pltpu.VMEM((CH,), jnp.int32),
            pltpu.VMEM((CH,), jnp.int32),
            pltpu.VMEM((CH, H), jnp.float32),
            pltpu.VMEM((CH, H), jnp.float32),
            pltpu.VMEM((CH, 16), jnp.float32),
            pltpu.VMEM((CH, HF), jnp.float32),
            pltpu.VMEM((CH, HF), jnp.float32),
            pltpu.VMEM((CH, F), jnp.float32),
            pltpu.VMEM_SHARED((NPAD, F), jnp.float32),
            pltpu.SemaphoreType.DMA,
            pltpu.SemaphoreType.DMA,
            pltpu.SemaphoreType.DMA,
            pltpu.SemaphoreType.DMA,
        ],
        compiler_params=_sc_compiler_params(),
    )(feat, src, dst, eexp, invd)


def _final_body(p_ref, b_ref, o_ref):
    sm = (p_ref[0] + p_ref[1]) * (1.0 / H)
    b = b_ref[...]
    mb = (b[:, 0:64] + b[:, 64:128] + b[:, 128:192] + b[:, 192:256]) * (1.0 / H)
    o_ref[...] = jnp.maximum(sm + mb, 0.0)


def _finalize(opart, bias2d):
    blk = 1000
    return pl.pallas_call(
        _final_body,
        grid=(N // blk,),
        in_specs=[
            pl.BlockSpec((NC, blk, F), lambda i: (0, i, 0)),
            pl.BlockSpec((1, HF), lambda i: (0, 0)),
        ],
        out_specs=pl.BlockSpec((blk, F), lambda i: (i, 0)),
        out_shape=jax.ShapeDtypeStruct((N, F), jnp.float32),
    )(opart, bias2d)


def kernel(x, edge_index, W, attn_l, attn_r, bias):
    src = edge_index[0].astype(jnp.int32)
    dst = edge_index[1].astype(jnp.int32)

    eye = jnp.eye(H, dtype=jnp.float32)
    Al = jnp.einsum("hf,hk->hfk", attn_l, eye).reshape(HF, H)
    Ar = jnp.einsum("hf,hk->hfk", attn_r, eye).reshape(HF, H)
    A_lr = jnp.concatenate(
        [Al, Ar, jnp.zeros((HF, 8), jnp.float32)], axis=1)

    feat, eler = _project(x, W, A_lr)
    eexp, dpart = _edge_logits(eler, src, dst)
    invd = _inv_denom(dpart[:, :N, :])
    opart = _aggregate(feat, src, dst, eexp, invd)
    return _finalize(opart[:, :N, :], bias.reshape(1, HF))
